# Initial kernel scaffold; baseline (speedup 1.0000x reference)
import jax
import jax.numpy as jnp
from jax.experimental import pallas as pl


def kernel(input_ids, mask, token_table, pos_table, mask_embedding, gamma, beta):
    raise NotImplementedError("write your pallas kernel here")


# speedup vs baseline: 1.8555x; 1.8555x over previous
import jax
import jax.numpy as jnp
from jax.experimental import pallas as pl
from jax.experimental.pallas import tpu as pltpu
from jax.experimental.pallas import tpu_sc as plsc


def _sc_gather(table, ids_flat, n, embed):
    window = 640
    mesh = plsc.VectorSubcoreMesh(core_axis_name="c", subcore_axis_name="s")

    @pl.kernel(
        out_type=jax.ShapeDtypeStruct((n, embed), jnp.float32),
        mesh=mesh,
        compiler_params=pltpu.CompilerParams(use_tc_tiling_on_sc=False),
    )
    def gather_kernel(table_hbm, ids_hbm, out_hbm):
        def body(i_vmem, o_vmem):
            pltpu.sync_copy(table_hbm.at[i_vmem.at[0]], o_vmem)

        pltpu.emit_pipeline(
            body,
            grid=(n // window,),
            in_specs=[pl.BlockSpec((1, window), lambda i: (0, i))],
            out_specs=[pl.BlockSpec((window, embed), lambda i: (i, 0))],
            core_axis_name=("c", "s"),
            dimension_semantics=(pltpu.PARALLEL,),
        )(ids_hbm, out_hbm)

    return gather_kernel(table, ids_flat)


def _tc_body(g_ref, m_ref, p_ref, me_ref, ga_ref, be_ref, o_ref):
    x = g_ref[...]
    m = m_ref[...]
    x = x * (1.0 - m) + me_ref[...] * m
    x = x + p_ref[...]
    mean = jnp.mean(x, axis=-1, keepdims=True)
    var = jnp.mean(jnp.square(x - mean), axis=-1, keepdims=True)
    o_ref[...] = (x - mean) * jax.lax.rsqrt(var + 1e-5) * ga_ref[...] + be_ref[...]


def kernel(input_ids, mask, token_table, pos_table, mask_embedding, gamma, beta):
    b, s = input_ids.shape
    vocab, embed = token_table.shape
    n = b * s

    ids_flat = input_ids.reshape(1, n).astype(jnp.int32)
    gathered = _sc_gather(token_table, ids_flat, n, embed)
    gathered = gathered.reshape(b, s, embed)

    mask_f = mask.astype(jnp.float32).reshape(b, s, 1)
    pos = pos_table[:s].reshape(1, s, embed)
    me = mask_embedding.reshape(1, 1, embed)
    ga = gamma.reshape(1, 1, embed)
    be = beta.reshape(1, 1, embed)

    bb = 16
    grid = (b // bb,)
    out = pl.pallas_call(
        _tc_body,
        grid=grid,
        in_specs=[
            pl.BlockSpec((bb, s, embed), lambda i: (i, 0, 0)),
            pl.BlockSpec((bb, s, 1), lambda i: (i, 0, 0)),
            pl.BlockSpec((1, s, embed), lambda i: (0, 0, 0)),
            pl.BlockSpec((1, 1, embed), lambda i: (0, 0, 0)),
            pl.BlockSpec((1, 1, embed), lambda i: (0, 0, 0)),
            pl.BlockSpec((1, 1, embed), lambda i: (0, 0, 0)),
        ],
        out_specs=pl.BlockSpec((bb, s, embed), lambda i: (i, 0, 0)),
        out_shape=jax.ShapeDtypeStruct((b, s, embed), jnp.float32),
    )(gathered, mask_f, pos, me, ga, be)
    return out
